# baseline (speedup 1.0000x reference)
import functools
import jax
import jax.numpy as jnp
from jax import lax
from jax.experimental import pallas as pl
from jax.experimental.pallas import tpu as pltpu
from jax.experimental.pallas import tpu_sc as plsc

_N = 16384
_M = 1000
_NL = 8192
_NW = 32
_RW = _NL // _NW
_OH = 32
_NOH = _RW // _OH
_CC = 16
_NCC = _RW // _CC
_NFULL = _M // 16
_TAIL0 = _NFULL * 16


def _body(probs_hbm, labs_hbm, out_hbm, labs_v, zb0, zb1, cb0, cb1,
          zs0, zs1, gs0, gs1, ss0, ss1):
    wid = lax.axis_index("c") * 16 + lax.axis_index("s")
    base = wid * _RW

    pltpu.sync_copy(labs_hbm.at[pl.ds(base, _RW)], labs_v)

    zeros16 = jnp.zeros((16,), jnp.float32)
    ones16 = jnp.ones((16,), jnp.float32)
    lane = lax.iota(jnp.int32, 16)

    zbufs = (zb0, zb1)
    zsems = (zs0, zs1)
    cbufs = (cb0, cb1)
    gsems = (gs0, gs1)
    ssems = (ss0, ss1)
    gathers = {}
    scatters = {}
    zcopies = {}

    def _start_gather(k):
        b = k % 2
        h = pltpu.make_async_copy(
            probs_hbm.at[pl.ds(_NL + base + k * _CC, _CC)], cbufs[b], gsems[b])
        h.start()
        gathers[k] = h

    _start_gather(0)
    _start_gather(1)

    def _zero_vec(r, carry):
        def _cols(c, carry2):
            zb0[r, pl.ds(c * 16, 16)] = zeros16
            zb1[r, pl.ds(c * 16, 16)] = zeros16
            return carry2
        return lax.fori_loop(0, _NFULL, _cols, carry)

    lax.fori_loop(0, _OH, _zero_vec, 0)
    for g in range(_OH // 16):
        rows = lane + g * 16
        for t in range(_M - _TAIL0):
            col = jnp.full((16,), _TAIL0 + t, jnp.int32)
            plsc.store_scatter(zb0, [rows, col], zeros16)
            plsc.store_scatter(zb1, [rows, col], zeros16)

    def _labs(s, g):
        return labs_v[pl.ds(s * _OH + g * 16, 16)]

    def _copy_step(k):
        b = k % 2
        gathers[k].wait()
        h = pltpu.make_async_copy(
            cbufs[b], out_hbm.at[pl.ds(_NL + base + k * _CC, _CC)], ssems[b])
        h.start()
        scatters[k] = h
        if k + 2 < _NCC:
            h.wait()
            _start_gather(k + 2)

    for s in range(_NOH):
        b = s % 2
        zbb = zbufs[b]
        if s >= 2:
            zcopies[s - 2].wait()
            for g in range(_OH // 16):
                plsc.store_scatter(zbb, [lane + g * 16, _labs(s - 2, g)],
                                   zeros16)
        for g in range(_OH // 16):
            plsc.store_scatter(zbb, [lane + g * 16, _labs(s, g)], ones16)
        h = pltpu.make_async_copy(
            zbb, out_hbm.at[pl.ds(base + s * _OH, _OH)], zsems[b])
        h.start()
        zcopies[s] = h
        _copy_step(2 * s)
        _copy_step(2 * s + 1)

    zcopies[_NOH - 2].wait()
    zcopies[_NOH - 1].wait()
    scatters[_NCC - 2].wait()
    scatters[_NCC - 1].wait()


def kernel(probs, labs, L, U):
    mesh = plsc.VectorSubcoreMesh(core_axis_name="c", subcore_axis_name="s")
    run = functools.partial(
        pl.kernel,
        mesh=mesh,
        compiler_params=pltpu.CompilerParams(
            needs_layout_passes=False, use_tc_tiling_on_sc=True),
        out_type=jax.ShapeDtypeStruct((_N, _M), jnp.float32),
        scratch_types=[
            pltpu.VMEM((_RW,), jnp.int32),
            pltpu.VMEM((_OH, _M), jnp.float32),
            pltpu.VMEM((_OH, _M), jnp.float32),
            pltpu.VMEM((_CC, _M), jnp.float32),
            pltpu.VMEM((_CC, _M), jnp.float32),
            pltpu.SemaphoreType.DMA,
            pltpu.SemaphoreType.DMA,
            pltpu.SemaphoreType.DMA,
            pltpu.SemaphoreType.DMA,
            pltpu.SemaphoreType.DMA,
            pltpu.SemaphoreType.DMA,
        ],
    )(_body)
    return run(probs, labs.astype(jnp.int32))


# speedup vs baseline: 1.0434x; 1.0434x over previous
import functools
import jax
import jax.numpy as jnp
from jax import lax
from jax.experimental import pallas as pl
from jax.experimental.pallas import tpu as pltpu
from jax.experimental.pallas import tpu_sc as plsc

_N = 16384
_M = 1000
_NL = 8192
_NW = 32
_RW = _NL // _NW
_CH = 32
_NCH = _RW // _CH
_NFULL = _M // 16
_TAIL0 = _NFULL * 16


def _body(probs_hbm, labs_hbm, out_hbm, labs_v, zb, cb0, cb1,
          gs0, gs1, ss0, ss1):
    wid = lax.axis_index("c") * 16 + lax.axis_index("s")
    base = wid * _RW

    pltpu.sync_copy(labs_hbm.at[pl.ds(base, _RW)], labs_v)

    zeros16 = jnp.zeros((16,), jnp.float32)
    ones16 = jnp.ones((16,), jnp.float32)
    lane = lax.iota(jnp.int32, 16)

    cbufs = (cb0, cb1)
    gsems = (gs0, gs1)
    ssems = (ss0, ss1)
    gathers = {}
    scatters = {}

    def _start_gather(k):
        b = k % 2
        h = pltpu.make_async_copy(
            probs_hbm.at[pl.ds(_NL + base + k * _CH, _CH)], cbufs[b], gsems[b])
        h.start()
        gathers[k] = h

    _start_gather(0)
    _start_gather(1)

    def _zero_row(r, carry):
        for c in range(_NFULL):
            zb[r, pl.ds(c * 16, 16)] = zeros16
        return carry

    lax.fori_loop(0, _CH, _zero_row, 0)
    for g in range(_CH // 16):
        rows = lane + g * 16
        for t in range(_M - _TAIL0):
            plsc.store_scatter(zb, [rows, jnp.full((16,), _TAIL0 + t, jnp.int32)],
                               zeros16)

    def _copy_step(k):
        b = k % 2
        gathers[k].wait()
        h = pltpu.make_async_copy(
            cbufs[b], out_hbm.at[pl.ds(_NL + base + k * _CH, _CH)], ssems[b])
        h.start()
        scatters[k] = h
        if k + 2 < _NCH:
            h.wait()
            _start_gather(k + 2)

    for s in range(_NCH):
        for g in range(_CH // 16):
            lab16 = labs_v[pl.ds(s * _CH + g * 16, 16)]
            plsc.store_scatter(zb, [lane + g * 16, lab16], ones16)
        pltpu.sync_copy(zb, out_hbm.at[pl.ds(base + s * _CH, _CH)])
        for g in range(_CH // 16):
            lab16 = labs_v[pl.ds(s * _CH + g * 16, 16)]
            plsc.store_scatter(zb, [lane + g * 16, lab16], zeros16)
        _copy_step(s)
    scatters[_NCH - 2].wait()
    scatters[_NCH - 1].wait()


def kernel(probs, labs, L, U):
    mesh = plsc.VectorSubcoreMesh(core_axis_name="c", subcore_axis_name="s")
    run = functools.partial(
        pl.kernel,
        mesh=mesh,
        compiler_params=pltpu.CompilerParams(
            needs_layout_passes=False, use_tc_tiling_on_sc=True),
        out_type=jax.ShapeDtypeStruct((_N, _M), jnp.float32),
        scratch_types=[
            pltpu.VMEM((_RW,), jnp.int32),
            pltpu.VMEM((_CH, _M), jnp.float32),
            pltpu.VMEM((_CH, _M), jnp.float32),
            pltpu.VMEM((_CH, _M), jnp.float32),
            pltpu.SemaphoreType.DMA,
            pltpu.SemaphoreType.DMA,
            pltpu.SemaphoreType.DMA,
            pltpu.SemaphoreType.DMA,
        ],
    )(_body)
    return run(probs, labs.astype(jnp.int32))


# speedup vs baseline: 1.0509x; 1.0072x over previous
import functools
import jax
import jax.numpy as jnp
from jax import lax
from jax.experimental import pallas as pl
from jax.experimental.pallas import tpu as pltpu
from jax.experimental.pallas import tpu_sc as plsc

_N = 16384
_M = 1000
_NL = 8192
_NW = 32
_RW = _NL // _NW
_CH = 32
_NCH = _RW // _CH
_NFULL = _M // 16
_TAIL0 = _NFULL * 16


def _body(probs_hbm, labs_hbm, out_hbm, labs_v, zb, cb0, cb1,
          gs0, gs1, ss0, ss1):
    wid = lax.axis_index("c") * 16 + lax.axis_index("s")
    base = wid * _RW

    pltpu.sync_copy(labs_hbm.at[pl.ds(base, _RW)], labs_v)

    zeros16 = jnp.zeros((16,), jnp.float32)
    ones16 = jnp.ones((16,), jnp.float32)
    lane = lax.iota(jnp.int32, 16)

    cbufs = (cb0, cb1)
    gsems = (gs0, gs1)
    ssems = (ss0, ss1)
    gathers = {}
    scatters = {}

    def _start_gather(k):
        b = k % 2
        h = pltpu.make_async_copy(
            probs_hbm.at[pl.ds(_NL + base + k * _CH, _CH)], cbufs[b], gsems[b])
        h.start()
        gathers[k] = h

    _start_gather(0)
    _start_gather(1)

    def _zero_row(r, carry):
        for c in range(_NFULL):
            zb[r, pl.ds(c * 16, 16)] = zeros16
        return carry

    lax.fori_loop(0, _CH, _zero_row, 0)
    for g in range(_CH // 16):
        rows = lane + g * 16
        for t in range(_M - _TAIL0):
            plsc.store_scatter(zb, [rows, jnp.full((16,), _TAIL0 + t, jnp.int32)],
                               zeros16)

    for s in range(_NCH):
        for g in range(_CH // 16):
            lab16 = labs_v[pl.ds(s * _CH + g * 16, 16)]
            plsc.store_scatter(zb, [lane + g * 16, lab16], ones16)
        pltpu.sync_copy(zb, out_hbm.at[pl.ds(base + s * _CH, _CH)])
        for g in range(_CH // 16):
            lab16 = labs_v[pl.ds(s * _CH + g * 16, 16)]
            plsc.store_scatter(zb, [lane + g * 16, lab16], zeros16)

    for k in range(_NCH):
        b = k % 2
        gathers[k].wait()
        h = pltpu.make_async_copy(
            cbufs[b], out_hbm.at[pl.ds(_NL + base + k * _CH, _CH)], ssems[b])
        h.start()
        scatters[k] = h
        if k + 2 < _NCH:
            h.wait()
            _start_gather(k + 2)
    scatters[_NCH - 2].wait()
    scatters[_NCH - 1].wait()


def kernel(probs, labs, L, U):
    mesh = plsc.VectorSubcoreMesh(core_axis_name="c", subcore_axis_name="s")
    run = functools.partial(
        pl.kernel,
        mesh=mesh,
        compiler_params=pltpu.CompilerParams(
            needs_layout_passes=False, use_tc_tiling_on_sc=True),
        out_type=jax.ShapeDtypeStruct((_N, _M), jnp.float32),
        scratch_types=[
            pltpu.VMEM((_RW,), jnp.int32),
            pltpu.VMEM((_CH, _M), jnp.float32),
            pltpu.VMEM((_CH, _M), jnp.float32),
            pltpu.VMEM((_CH, _M), jnp.float32),
            pltpu.SemaphoreType.DMA,
            pltpu.SemaphoreType.DMA,
            pltpu.SemaphoreType.DMA,
            pltpu.SemaphoreType.DMA,
        ],
    )(_body)
    return run(probs, labs.astype(jnp.int32))
